# Initial kernel scaffold; baseline (speedup 1.0000x reference)
import jax
import jax.numpy as jnp
from jax.experimental import pallas as pl


def kernel(x, edge_index, W1, b1, W2, b2):
    raise NotImplementedError("write your pallas kernel here")


# speedup vs baseline: 23.8265x; 23.8265x over previous
import functools

import jax
import jax.numpy as jnp
from jax import lax
from jax.experimental import pallas as pl
from jax.experimental.pallas import tpu as pltpu
from jax.experimental.pallas import tpu_sc as plsc

N = 10000
F = 128
E = 320000
NC, NS = 2, 16
NW = NC * NS
CHUNK = 128
CPT = 80
EP = NW * CPT * CHUNK
NP = 10240
RPT = NP // NS
NZC = RPT // 128

@functools.cache
def _sc_mesh():
    return plsc.VectorSubcoreMesh(core_axis_name="c", subcore_axis_name="s",
                                  num_cores=NC, num_subcores=NS)


def _deg_body(dst_hbm, ones_hbm, zeros_hbm, out_hbm, idx_v, ones_v, zbuf,
              deg_sp):
    c = lax.axis_index("c")
    s = lax.axis_index("s")
    w = c * NS + s
    pltpu.sync_copy(zeros_hbm, zbuf)
    for k in range(NZC):
        pltpu.sync_copy(zbuf, deg_sp.at[pl.ds(s * RPT + k * 128, 128)])
    pltpu.sync_copy(ones_hbm, ones_v)
    pltpu.sync_copy(dst_hbm.at[w], idx_v)
    plsc.subcore_barrier()

    @pl.loop(0, CPT)
    def _(j):
        pltpu.sync_copy(ones_v, deg_sp.at[idx_v.at[j]], add=True)

    plsc.subcore_barrier()
    for k in range(NZC):
        r = s * RPT + k * 128
        pltpu.sync_copy(deg_sp.at[pl.ds(r, 128)],
                        out_hbm.at[pl.ds(c * NP + r, 128)])


@functools.cache
def _deg_call():
    return pl.kernel(
        _deg_body,
        out_type=jax.ShapeDtypeStruct((NC * NP, 16), jnp.float32),
        mesh=_sc_mesh(),
        scratch_types=[
            pltpu.VMEM((CPT, CHUNK), jnp.int32),
            pltpu.VMEM((CHUNK, 16), jnp.float32),
            pltpu.VMEM((CHUNK, 16), jnp.float32),
            pltpu.VMEM_SHARED((NP, 16), jnp.float32),
        ],
    )


SB = 8
NSUP = CPT // SB


def _agg_body(g_hbm, src_hbm, dst_hbm, zeros_hbm, out_hbm,
              src_s0, src_s1, dst_s0, dst_s1, buf0, buf1, acc_sp,
              sem0, sem1, semi0, semi1):
    c = lax.axis_index("c")
    s = lax.axis_index("s")
    w = c * NS + s
    pltpu.sync_copy(zeros_hbm, buf0)
    for k in range(NZC):
        pltpu.sync_copy(buf0, acc_sp.at[pl.ds(s * RPT + k * 128, 128)])
    plsc.subcore_barrier()

    src_sl = (src_s0, src_s1)
    dst_sl = (dst_s0, dst_s1)
    semi = (semi0, semi1)
    for i in range(NSUP):
        sl = i % 2
        if i == 0:
            pltpu.sync_copy(src_hbm.at[w * NSUP], src_s0)
            pltpu.sync_copy(dst_hbm.at[w * NSUP], dst_s0)
        else:
            pltpu.make_async_copy(src_hbm.at[w * NSUP + i], src_sl[sl],
                                  semi[sl]).wait()
            pltpu.make_async_copy(dst_hbm.at[w * NSUP + i], dst_sl[sl],
                                  semi[sl]).wait()
        if i + 1 < NSUP:
            nsl = (i + 1) % 2
            pltpu.async_copy(src_hbm.at[w * NSUP + i + 1], src_sl[nsl],
                             semi[nsl])
            pltpu.async_copy(dst_hbm.at[w * NSUP + i + 1], dst_sl[nsl],
                             semi[nsl])
        sv, dv = src_sl[sl], dst_sl[sl]
        pltpu.async_copy(g_hbm.at[sv.at[0]], buf0, sem0)

        @pl.loop(0, SB // 2)
        def _(t):
            j0 = t * 2
            pltpu.async_copy(g_hbm.at[sv.at[j0 + 1]], buf1, sem1)
            pltpu.make_async_copy(g_hbm.at[sv.at[j0]], buf0, sem0).wait()
            pltpu.sync_copy(buf0, acc_sp.at[dv.at[j0]], add=True)

            @pl.when(t < SB // 2 - 1)
            def _():
                pltpu.async_copy(g_hbm.at[sv.at[j0 + 2]], buf0, sem0)

            pltpu.make_async_copy(g_hbm.at[sv.at[j0 + 1]], buf1, sem1).wait()
            pltpu.sync_copy(buf1, acc_sp.at[dv.at[j0 + 1]], add=True)

    plsc.subcore_barrier()
    for k in range(NZC):
        r = s * RPT + k * 128
        pltpu.sync_copy(acc_sp.at[pl.ds(r, 128)],
                        out_hbm.at[pl.ds(c * NP + r, 128)])


@functools.cache
def _agg_call():
    return pl.kernel(
        _agg_body,
        out_type=jax.ShapeDtypeStruct((NC * NP, F), jnp.float32),
        mesh=_sc_mesh(),
        scratch_types=[
            pltpu.VMEM((SB, CHUNK), jnp.int32),
            pltpu.VMEM((SB, CHUNK), jnp.int32),
            pltpu.VMEM((SB, CHUNK), jnp.int32),
            pltpu.VMEM((SB, CHUNK), jnp.int32),
            pltpu.VMEM((CHUNK, F), jnp.float32),
            pltpu.VMEM((CHUNK, F), jnp.float32),
            pltpu.VMEM_SHARED((NP, F), jnp.float32),
            pltpu.SemaphoreType.DMA,
            pltpu.SemaphoreType.DMA,
            pltpu.SemaphoreType.DMA,
            pltpu.SemaphoreType.DMA,
        ],
    )


_BM = 256
_GRID = (NP // _BM,)


def _dis(d0_ref, d1_ref):
    return lax.rsqrt(d0_ref[:, 0:1] + d1_ref[:, 0:1] + 1.0)


def _mm_scale_body(x_ref, w_ref, d0_ref, d1_ref, o_ref):
    dis = _dis(d0_ref, d1_ref)
    o_ref[...] = jnp.dot(x_ref[...], w_ref[...],
                         preferred_element_type=jnp.float32) * dis


def _combine_mm_body(a0_ref, a1_ref, g_ref, d0_ref, d1_ref, b_ref, w_ref,
                     o_ref):
    dis = _dis(d0_ref, d1_ref)
    r = jnp.maximum(dis * (a0_ref[...] + a1_ref[...] + g_ref[...])
                    + b_ref[...], 0.0)
    o_ref[...] = jnp.dot(r, w_ref[...],
                         preferred_element_type=jnp.float32) * dis


def _combine_relu_body(a0_ref, a1_ref, g_ref, d0_ref, d1_ref, b_ref, o_ref):
    dis = _dis(d0_ref, d1_ref)
    o_ref[...] = jnp.maximum(dis * (a0_ref[...] + a1_ref[...] + g_ref[...])
                             + b_ref[...], 0.0)


_row_spec = pl.BlockSpec((_BM, F), lambda i: (i, 0))
_deg_spec = pl.BlockSpec((_BM, 16), lambda i: (i, 0))
_w_spec = pl.BlockSpec((F, F), lambda i: (0, 0))
_b_spec = pl.BlockSpec((1, F), lambda i: (0, 0))
_out_t = jax.ShapeDtypeStruct((NP, F), jnp.float32)

_mm_scale = pl.pallas_call(
    _mm_scale_body,
    grid=_GRID,
    in_specs=[_row_spec, _w_spec, _deg_spec, _deg_spec],
    out_specs=_row_spec,
    out_shape=_out_t,
)

_combine_mm = pl.pallas_call(
    _combine_mm_body,
    grid=_GRID,
    in_specs=[_row_spec, _row_spec, _row_spec, _deg_spec, _deg_spec,
              _b_spec, _w_spec],
    out_specs=_row_spec,
    out_shape=_out_t,
)

_combine_relu = pl.pallas_call(
    _combine_relu_body,
    grid=_GRID,
    in_specs=[_row_spec, _row_spec, _row_spec, _deg_spec, _deg_spec, _b_spec],
    out_specs=_row_spec,
    out_shape=_out_t,
)


def kernel(x, edge_index, W1, b1, W2, b2):
    npad = EP - E
    padv = (N + (jnp.arange(npad, dtype=jnp.int32) % (NP - N)))
    src = jnp.concatenate([edge_index[0].astype(jnp.int32), padv])
    dst = jnp.concatenate([edge_index[1].astype(jnp.int32), padv])
    src = src.reshape(NW * NSUP, SB, CHUNK)
    dst = dst.reshape(NW * NSUP, SB, CHUNK)
    xp = jnp.pad(x, ((0, NP - N), (0, 0)))
    zeros16 = jnp.zeros((CHUNK, 16), jnp.float32)
    ones16 = jnp.ones((CHUNK, 16), jnp.float32)
    zeros128 = jnp.zeros((CHUNK, F), jnp.float32)
    b1r = b1.reshape(1, F)
    b2r = b2.reshape(1, F)

    deg = _deg_call()(dst.reshape(NW, CPT, CHUNK), ones16, zeros16)
    d0, d1 = deg[:NP], deg[NP:]

    g1 = _mm_scale(xp, W1, d0, d1)
    acc1 = _agg_call()(g1, src, dst, zeros128)
    g2 = _combine_mm(acc1[:NP], acc1[NP:], g1, d0, d1, b1r, W2)
    acc2 = _agg_call()(g2, src, dst, zeros128)
    out = _combine_relu(acc2[:NP], acc2[NP:], g2, d0, d1, b2r)
    return out[:N]


# speedup vs baseline: 25.2200x; 1.0585x over previous
import functools

import jax
import jax.numpy as jnp
from jax import lax
from jax.experimental import pallas as pl
from jax.experimental.pallas import tpu as pltpu
from jax.experimental.pallas import tpu_sc as plsc

N = 10000
F = 128
E = 320000
NC, NS = 2, 16
NW = NC * NS
CHUNK = 128
CPT = 80
EP = NW * CPT * CHUNK
NP = 10240
RPT = NP // NS
NZC = RPT // 128

@functools.cache
def _sc_mesh():
    return plsc.VectorSubcoreMesh(core_axis_name="c", subcore_axis_name="s",
                                  num_cores=NC, num_subcores=NS)


def _deg_body(dst_hbm, ones_hbm, zeros_hbm, out_hbm, idx_v, ones_v, zbuf,
              deg_sp, sem):
    c = lax.axis_index("c")
    s = lax.axis_index("s")
    w = c * NS + s
    pltpu.sync_copy(zeros_hbm, zbuf)
    for k in range(NZC):
        pltpu.sync_copy(zbuf, deg_sp.at[pl.ds(s * RPT + k * 128, 128)])
    pltpu.sync_copy(ones_hbm, ones_v)
    pltpu.sync_copy(dst_hbm.at[w], idx_v)
    plsc.subcore_barrier()

    @pl.loop(0, CPT)
    def _(j):
        pltpu.async_copy(ones_v, deg_sp.at[idx_v.at[j]], sem, add=True)

    @pl.loop(0, CPT)
    def _(j):
        pltpu.make_async_copy(ones_v, deg_sp.at[idx_v.at[j]], sem).wait()

    plsc.subcore_barrier()
    for k in range(NZC):
        r = s * RPT + k * 128
        pltpu.sync_copy(deg_sp.at[pl.ds(r, 128)],
                        out_hbm.at[pl.ds(c * NP + r, 128)])


@functools.cache
def _deg_call():
    return pl.kernel(
        _deg_body,
        out_type=jax.ShapeDtypeStruct((NC * NP, 16), jnp.float32),
        mesh=_sc_mesh(),
        scratch_types=[
            pltpu.VMEM((CPT, CHUNK), jnp.int32),
            pltpu.VMEM((CHUNK, 16), jnp.float32),
            pltpu.VMEM((CHUNK, 16), jnp.float32),
            pltpu.VMEM_SHARED((NP, 16), jnp.float32),
            pltpu.SemaphoreType.DMA,
        ],
    )


SB = 8
NSUP = CPT // SB


def _agg_body(g_hbm, src_hbm, dst_hbm, zeros_hbm, out_hbm,
              src_s0, src_s1, dst_s0, dst_s1, buf0, buf1, acc_sp,
              sem0, sem1, semi0, semi1):
    c = lax.axis_index("c")
    s = lax.axis_index("s")
    w = c * NS + s
    pltpu.sync_copy(zeros_hbm, buf0)
    for k in range(NZC):
        pltpu.sync_copy(buf0, acc_sp.at[pl.ds(s * RPT + k * 128, 128)])
    plsc.subcore_barrier()

    src_sl = (src_s0, src_s1)
    dst_sl = (dst_s0, dst_s1)
    semi = (semi0, semi1)
    for i in range(NSUP):
        sl = i % 2
        if i == 0:
            pltpu.sync_copy(src_hbm.at[w * NSUP], src_s0)
            pltpu.sync_copy(dst_hbm.at[w * NSUP], dst_s0)
        else:
            pltpu.make_async_copy(src_hbm.at[w * NSUP + i], src_sl[sl],
                                  semi[sl]).wait()
            pltpu.make_async_copy(dst_hbm.at[w * NSUP + i], dst_sl[sl],
                                  semi[sl]).wait()
        if i + 1 < NSUP:
            nsl = (i + 1) % 2
            pltpu.async_copy(src_hbm.at[w * NSUP + i + 1], src_sl[nsl],
                             semi[nsl])
            pltpu.async_copy(dst_hbm.at[w * NSUP + i + 1], dst_sl[nsl],
                             semi[nsl])
        sv, dv = src_sl[sl], dst_sl[sl]
        pltpu.async_copy(g_hbm.at[sv.at[0]], buf0, sem0)

        @pl.loop(0, SB // 2)
        def _(t):
            j0 = t * 2
            pltpu.async_copy(g_hbm.at[sv.at[j0 + 1]], buf1, sem1)
            pltpu.make_async_copy(g_hbm.at[sv.at[j0]], buf0, sem0).wait()
            pltpu.sync_copy(buf0, acc_sp.at[dv.at[j0]], add=True)

            @pl.when(t < SB // 2 - 1)
            def _():
                pltpu.async_copy(g_hbm.at[sv.at[j0 + 2]], buf0, sem0)

            pltpu.make_async_copy(g_hbm.at[sv.at[j0 + 1]], buf1, sem1).wait()
            pltpu.sync_copy(buf1, acc_sp.at[dv.at[j0 + 1]], add=True)

    plsc.subcore_barrier()
    for k in range(NZC):
        r = s * RPT + k * 128
        pltpu.sync_copy(acc_sp.at[pl.ds(r, 128)],
                        out_hbm.at[pl.ds(c * NP + r, 128)])


@functools.cache
def _agg_call():
    return pl.kernel(
        _agg_body,
        out_type=jax.ShapeDtypeStruct((NC * NP, F), jnp.float32),
        mesh=_sc_mesh(),
        scratch_types=[
            pltpu.VMEM((SB, CHUNK), jnp.int32),
            pltpu.VMEM((SB, CHUNK), jnp.int32),
            pltpu.VMEM((SB, CHUNK), jnp.int32),
            pltpu.VMEM((SB, CHUNK), jnp.int32),
            pltpu.VMEM((CHUNK, F), jnp.float32),
            pltpu.VMEM((CHUNK, F), jnp.float32),
            pltpu.VMEM_SHARED((NP, F), jnp.float32),
            pltpu.SemaphoreType.DMA,
            pltpu.SemaphoreType.DMA,
            pltpu.SemaphoreType.DMA,
            pltpu.SemaphoreType.DMA,
        ],
    )


_BM = 256
_GRID = (NP // _BM,)


def _dis(d0_ref, d1_ref):
    return lax.rsqrt(d0_ref[:, 0:1] + d1_ref[:, 0:1] + 1.0)


def _mm_scale_body(x_ref, w_ref, d0_ref, d1_ref, o_ref):
    dis = _dis(d0_ref, d1_ref)
    o_ref[...] = jnp.dot(x_ref[...], w_ref[...],
                         preferred_element_type=jnp.float32) * dis


def _combine_mm_body(a0_ref, a1_ref, g_ref, d0_ref, d1_ref, b_ref, w_ref,
                     o_ref):
    dis = _dis(d0_ref, d1_ref)
    r = jnp.maximum(dis * (a0_ref[...] + a1_ref[...] + g_ref[...])
                    + b_ref[...], 0.0)
    o_ref[...] = jnp.dot(r, w_ref[...],
                         preferred_element_type=jnp.float32) * dis


def _combine_relu_body(a0_ref, a1_ref, g_ref, d0_ref, d1_ref, b_ref, o_ref):
    dis = _dis(d0_ref, d1_ref)
    o_ref[...] = jnp.maximum(dis * (a0_ref[...] + a1_ref[...] + g_ref[...])
                             + b_ref[...], 0.0)


_NB = NP // _BM
_row_spec = pl.BlockSpec((_BM, F), lambda i: (i, 0))
_row_spec_hi = pl.BlockSpec((_BM, F), lambda i: (i + _NB, 0))
_deg_spec = pl.BlockSpec((_BM, 16), lambda i: (i, 0))
_deg_spec_hi = pl.BlockSpec((_BM, 16), lambda i: (i + _NB, 0))
_w_spec = pl.BlockSpec((F, F), lambda i: (0, 0))
_b_spec = pl.BlockSpec((1, F), lambda i: (0, 0))
_out_t = jax.ShapeDtypeStruct((NP, F), jnp.float32)

_mm_scale = pl.pallas_call(
    _mm_scale_body,
    grid=_GRID,
    in_specs=[_row_spec, _w_spec, _deg_spec, _deg_spec_hi],
    out_specs=_row_spec,
    out_shape=_out_t,
)

_combine_mm = pl.pallas_call(
    _combine_mm_body,
    grid=_GRID,
    in_specs=[_row_spec, _row_spec_hi, _row_spec, _deg_spec, _deg_spec_hi,
              _b_spec, _w_spec],
    out_specs=_row_spec,
    out_shape=_out_t,
)

_combine_relu = pl.pallas_call(
    _combine_relu_body,
    grid=_GRID,
    in_specs=[_row_spec, _row_spec_hi, _row_spec, _deg_spec, _deg_spec_hi,
              _b_spec],
    out_specs=_row_spec,
    out_shape=_out_t,
)


def kernel(x, edge_index, W1, b1, W2, b2):
    npad = EP - E
    padv = (N + (jnp.arange(npad, dtype=jnp.int32) % (NP - N)))
    src = jnp.concatenate([edge_index[0].astype(jnp.int32), padv])
    dst = jnp.concatenate([edge_index[1].astype(jnp.int32), padv])
    src = src.reshape(NW * NSUP, SB, CHUNK)
    dst = dst.reshape(NW * NSUP, SB, CHUNK)
    xp = jnp.pad(x, ((0, NP - N), (0, 0)))
    zeros16 = jnp.zeros((CHUNK, 16), jnp.float32)
    ones16 = jnp.ones((CHUNK, 16), jnp.float32)
    zeros128 = jnp.zeros((CHUNK, F), jnp.float32)
    b1r = b1.reshape(1, F)
    b2r = b2.reshape(1, F)

    deg = _deg_call()(dst.reshape(NW, CPT, CHUNK), ones16, zeros16)

    g1 = _mm_scale(xp, W1, deg, deg)
    acc1 = _agg_call()(g1, src, dst, zeros128)
    g2 = _combine_mm(acc1, acc1, g1, deg, deg, b1r, W2)
    acc2 = _agg_call()(g2, src, dst, zeros128)
    out = _combine_relu(acc2, acc2, g2, deg, deg, b2r)
    return out[:N]
